# baseline (speedup 1.0000x reference)
import functools

import jax
import jax.numpy as jnp
from jax import lax
from jax.experimental import pallas as pl
from jax.experimental.pallas import tpu as pltpu
from jax.experimental.pallas import tpu_sc as plsc

_VOCAB = 1000000
_D = 64
_B = 16384
_L = 50
_COEFF = 0.5
_N = _B * _L


_CB = 8192


def _merge_body(t0_ref, t1_ref, w_ref, out_ref):
    a = jnp.concatenate([t0_ref[...], t1_ref[...]], axis=0)
    out_ref[:, : _D] = lax.dot_general(
        a, w_ref[...], (((0,), (0,)), ((), ())),
        preferred_element_type=jnp.float32,
    )


def _merge_tables(t0t, t1t, w):
    grid = (pl.cdiv(_VOCAB, _CB),)
    return pl.pallas_call(
        _merge_body,
        grid=grid,
        in_specs=[
            pl.BlockSpec((_D, _CB), lambda i: (0, i)),
            pl.BlockSpec((_D, _CB), lambda i: (0, i)),
            pl.BlockSpec((2 * _D, _D), lambda i: (0, 0)),
        ],
        out_specs=pl.BlockSpec((_CB, 2 * _D), lambda i: (i, 0)),
        out_shape=jax.ShapeDtypeStruct((_VOCAB, 2 * _D), jnp.float32),
    )(t0t, t1t, w)


_NW = 32
_BW = _B // _NW
_G = 128
_NG = _BW // _G


def _gather_merged(merged2, idx3):
    mesh = plsc.VectorSubcoreMesh(core_axis_name="c", subcore_axis_name="s")

    @functools.partial(
        pl.kernel,
        mesh=mesh,
        out_type=jax.ShapeDtypeStruct((_N, _D), jnp.float32),
        scratch_types=[
            pltpu.VMEM((_NG, _G), jnp.int32),
            pltpu.VMEM((_BW, _D), jnp.float32),
            pltpu.SemaphoreType.DMA,
        ],
        compiler_params=pltpu.CompilerParams(use_tc_tiling_on_sc=False),
    )
    def _gather_kernel(m_hbm, idx_hbm, out_hbm, idx_v, rows_v, sem):
        wid = lax.axis_index("s") * 2 + lax.axis_index("c")
        b0 = wid * _BW

        def body(l, carry):
            pltpu.sync_copy(idx_hbm.at[l, pl.ds(wid * _NG, _NG)], idx_v)
            copies = []
            for j in range(_NG):
                copies.append(
                    pltpu.async_copy(
                        m_hbm.at[idx_v.at[j]],
                        rows_v.at[pl.ds(j * _G, _G)],
                        sem,
                    )
                )
            for c in copies:
                c.wait()
            pltpu.sync_copy(rows_v, out_hbm.at[pl.ds(l * _B + b0, _BW)])
            return carry

        lax.fori_loop(0, _L, body, 0)

    return _gather_kernel(merged2, idx3)


_HB = _BW // 2


def _format_body(g_ref, out_ref):
    t = g_ref[0, 0].T
    out_ref[0, :, : _HB] = t[: _D]
    out_ref[0, :, _HB:] = t[_D:]


def _format_out(g4):
    grid = (_L, _B // _BW)
    return pl.pallas_call(
        _format_body,
        grid=grid,
        in_specs=[pl.BlockSpec((1, 1, _HB, 2 * _D), lambda l, c: (l, c, 0, 0))],
        out_specs=pl.BlockSpec((1, _D, _BW), lambda l, c: (l, 0, c)),
        out_shape=jax.ShapeDtypeStruct((_L, _D, _B), jnp.float32),
    )(g4)


def kernel(x, table0, table1, P):
    w = jnp.concatenate(
        [(1.0 - _COEFF) * jnp.eye(_D, dtype=jnp.float32), _COEFF * P], axis=0
    )
    merged = _merge_tables(table0.T, table1.T, w)
    idx = 2 * x.T.astype(jnp.int32)
    idx_perm = idx.reshape(_L, _B // _BW, 2, _HB).transpose(0, 1, 3, 2)
    idx3 = idx_perm.reshape(_L, _B // _G, _G)
    g = _gather_merged(merged.reshape(2 * _VOCAB, _D), idx3)
    out = _format_out(g.reshape(_L, _B // _BW, _HB, 2 * _D))
    return jnp.transpose(out, (2, 0, 1))


# speedup vs baseline: 2.0608x; 2.0608x over previous
import functools

import jax
import jax.numpy as jnp
from jax import lax
from jax.experimental import pallas as pl
from jax.experimental.pallas import tpu as pltpu
from jax.experimental.pallas import tpu_sc as plsc

_VOCAB = 1000000
_D = 64
_B = 16384
_L = 50
_COEFF = 0.5
_N = _B * _L


_CB = 8192


def _merge_body(t0_ref, t1_ref, w_ref, out_ref):
    a = jnp.concatenate([t0_ref[...], t1_ref[...]], axis=0)
    out_ref[:, : _D] = lax.dot_general(
        a, w_ref[...], (((0,), (0,)), ((), ())),
        preferred_element_type=jnp.float32,
    )


def _merge_tables(t0t, t1t, w):
    grid = (pl.cdiv(_VOCAB, _CB),)
    return pl.pallas_call(
        _merge_body,
        grid=grid,
        in_specs=[
            pl.BlockSpec((_D, _CB), lambda i: (0, i)),
            pl.BlockSpec((_D, _CB), lambda i: (0, i)),
            pl.BlockSpec((2 * _D, _D), lambda i: (0, 0)),
        ],
        out_specs=pl.BlockSpec((_CB, 2 * _D), lambda i: (i, 0)),
        out_shape=jax.ShapeDtypeStruct((_VOCAB, 2 * _D), jnp.float32),
    )(t0t, t1t, w)


_NW = 32
_BW = _B // _NW
_G = 128
_NG = _BW // _G


def _gather_merged(merged2, idx3):
    mesh = plsc.VectorSubcoreMesh(core_axis_name="c", subcore_axis_name="s")

    @functools.partial(
        pl.kernel,
        mesh=mesh,
        out_type=jax.ShapeDtypeStruct((_N, _D), jnp.float32),
        scratch_types=[
            pltpu.VMEM((_NG, _G), jnp.int32),
            pltpu.VMEM((_BW, _D), jnp.float32),
            pltpu.SemaphoreType.DMA,
        ],
        compiler_params=pltpu.CompilerParams(use_tc_tiling_on_sc=False),
    )
    def _gather_kernel(m_hbm, idx_hbm, out_hbm, idx_v, rows_v, sem):
        wid = lax.axis_index("s") * 2 + lax.axis_index("c")
        b0 = wid * _BW

        def body(l, carry):
            pltpu.sync_copy(idx_hbm.at[l, pl.ds(wid * _NG, _NG)], idx_v)
            copies = []
            for j in range(_NG):
                copies.append(
                    pltpu.async_copy(
                        m_hbm.at[idx_v.at[j]],
                        rows_v.at[pl.ds(j * _G, _G)],
                        sem,
                    )
                )
            for c in copies:
                c.wait()
            pltpu.sync_copy(rows_v, out_hbm.at[pl.ds(l * _B + b0, _BW)])
            return carry

        lax.fori_loop(0, _L, body, 0)

    return _gather_kernel(merged2, idx3)


_HB = _BW // 2
_FC = 16


def _format_body(g_ref, out_ref):
    for c in range(_FC):
        t = g_ref[0, c].T
        out_ref[0, :, c * _BW : c * _BW + _HB] = t[: _D]
        out_ref[0, :, c * _BW + _HB : (c + 1) * _BW] = t[_D:]


def _format_out(g4):
    grid = (_L, _B // (_BW * _FC))
    return pl.pallas_call(
        _format_body,
        grid=grid,
        in_specs=[pl.BlockSpec((1, _FC, _HB, 2 * _D), lambda l, c: (l, c, 0, 0))],
        out_specs=pl.BlockSpec((1, _D, _BW * _FC), lambda l, c: (l, 0, c)),
        out_shape=jax.ShapeDtypeStruct((_L, _D, _B), jnp.float32),
    )(g4)


def kernel(x, table0, table1, P):
    w = jnp.concatenate(
        [(1.0 - _COEFF) * jnp.eye(_D, dtype=jnp.float32), _COEFF * P], axis=0
    )
    merged = _merge_tables(table0.T, table1.T, w)
    idx = 2 * x.T.astype(jnp.int32)
    idx_perm = idx.reshape(_L, _B // _BW, 2, _HB).transpose(0, 1, 3, 2)
    idx3 = idx_perm.reshape(_L, _B // _G, _G)
    g = _gather_merged(merged.reshape(2 * _VOCAB, _D), idx3)
    out = _format_out(g.reshape(_L, _B // _BW, _HB, 2 * _D))
    return jnp.transpose(out, (2, 0, 1))


# speedup vs baseline: 2.2455x; 1.0897x over previous
import functools

import jax
import jax.numpy as jnp
from jax import lax
from jax.experimental import pallas as pl
from jax.experimental.pallas import tpu as pltpu
from jax.experimental.pallas import tpu_sc as plsc

_VOCAB = 1000000
_D = 64
_B = 16384
_L = 50
_COEFF = 0.5
_N = _B * _L


_HALF = 524288
_CB = 8192
_NMB = _HALF // _CB


def _merge_body(t0lo_ref, t1lo_ref, t0hi_ref, t1hi_ref, w_ref, out_ref):
    alo = jnp.concatenate([t0lo_ref[...], t1lo_ref[...]], axis=0)
    ahi = jnp.concatenate([t0hi_ref[...], t1hi_ref[...]], axis=0)
    dn = (((0,), (0,)), ((), ()))
    out_ref[:, : _D] = lax.dot_general(
        alo, w_ref[...], dn, preferred_element_type=jnp.float32)
    out_ref[:, _D :] = lax.dot_general(
        ahi, w_ref[...], dn, preferred_element_type=jnp.float32)


def _merge_tables(t0t, t1t, w):
    last = pl.cdiv(_VOCAB, _CB) - 1
    lo = lambda i: (0, i)
    hi = lambda i: (0, jnp.minimum(i + _NMB, last))
    return pl.pallas_call(
        _merge_body,
        grid=(_NMB,),
        in_specs=[
            pl.BlockSpec((_D, _CB), lo),
            pl.BlockSpec((_D, _CB), lo),
            pl.BlockSpec((_D, _CB), hi),
            pl.BlockSpec((_D, _CB), hi),
            pl.BlockSpec((2 * _D, _D), lambda i: (0, 0)),
        ],
        out_specs=pl.BlockSpec((_CB, 2 * _D), lambda i: (i, 0)),
        out_shape=jax.ShapeDtypeStruct((_HALF, 2 * _D), jnp.float32),
    )(t0t, t1t, t0t, t1t, w)


_NW = 32
_BW = _B // _NW
_G = 128
_NG = _BW // _G


def _gather_merged(merged2, idx3):
    mesh = plsc.VectorSubcoreMesh(core_axis_name="c", subcore_axis_name="s")

    @functools.partial(
        pl.kernel,
        mesh=mesh,
        out_type=jax.ShapeDtypeStruct((_N, _D), jnp.float32),
        scratch_types=[
            pltpu.VMEM((_NG, _G), jnp.int32),
            pltpu.VMEM((_BW, _D), jnp.float32),
            pltpu.SemaphoreType.DMA,
        ],
        compiler_params=pltpu.CompilerParams(use_tc_tiling_on_sc=False),
    )
    def _gather_kernel(m_hbm, idx_hbm, out_hbm, idx_v, rows_v, sem):
        wid = lax.axis_index("s") * 2 + lax.axis_index("c")
        b0 = wid * _BW

        def body(l, carry):
            pltpu.sync_copy(idx_hbm.at[l, pl.ds(wid * _NG, _NG)], idx_v)
            copies = []
            for j in range(_NG):
                copies.append(
                    pltpu.async_copy(
                        m_hbm.at[idx_v.at[j]],
                        rows_v.at[pl.ds(j * _G, _G)],
                        sem,
                    )
                )
            for c in copies:
                c.wait()
            pltpu.sync_copy(rows_v, out_hbm.at[pl.ds(l * _B + b0, _BW)])
            return carry

        lax.fori_loop(0, _L, body, 0)

    return _gather_kernel(merged2, idx3)


_HB = _BW // 2
_FC = 16


def _format_body(g_ref, out_ref):
    for c in range(_FC):
        t = g_ref[0, c].T
        out_ref[0, :, c * _BW : c * _BW + _HB] = t[: _D]
        out_ref[0, :, c * _BW + _HB : (c + 1) * _BW] = t[_D:]


def _format_out(g4):
    grid = (_L, _B // (_BW * _FC))
    return pl.pallas_call(
        _format_body,
        grid=grid,
        in_specs=[pl.BlockSpec((1, _FC, _HB, 2 * _D), lambda l, c: (l, c, 0, 0))],
        out_specs=pl.BlockSpec((1, _D, _BW * _FC), lambda l, c: (l, 0, c)),
        out_shape=jax.ShapeDtypeStruct((_L, _D, _B), jnp.float32),
    )(g4)


def kernel(x, table0, table1, P):
    w = jnp.concatenate(
        [(1.0 - _COEFF) * jnp.eye(_D, dtype=jnp.float32), _COEFF * P], axis=0
    )
    merged = _merge_tables(table0.T, table1.T, w)
    xt = x.T.astype(jnp.int32)
    idx = jnp.where(xt < _HALF, 2 * xt, 2 * (xt - _HALF) + 1)
    idx_perm = idx.reshape(_L, _B // _BW, 2, _HB).transpose(0, 1, 3, 2)
    idx3 = idx_perm.reshape(_L, _B // _G, _G)
    g = _gather_merged(merged.reshape(2 * _HALF, _D), idx3)
    out = _format_out(g.reshape(_L, _B // _BW, _HB, 2 * _D))
    return jnp.transpose(out, (2, 0, 1))


# speedup vs baseline: 2.3558x; 1.0491x over previous
import functools

import jax
import jax.numpy as jnp
from jax import lax
from jax.experimental import pallas as pl
from jax.experimental.pallas import tpu as pltpu
from jax.experimental.pallas import tpu_sc as plsc

_VOCAB = 1000000
_D = 64
_B = 16384
_L = 50
_COEFF = 0.5
_N = _B * _L


_HALF = 524288
_CB = 8192
_NMB = _HALF // _CB


def _merge_body(t0lo_ref, t1lo_ref, t0hi_ref, t1hi_ref, w_ref, out_ref):
    alo = jnp.concatenate([t0lo_ref[...], t1lo_ref[...]], axis=0)
    ahi = jnp.concatenate([t0hi_ref[...], t1hi_ref[...]], axis=0)
    dn = (((0,), (0,)), ((), ()))
    out_ref[:, : _D] = lax.dot_general(
        alo, w_ref[...], dn, preferred_element_type=jnp.float32)
    out_ref[:, _D :] = lax.dot_general(
        ahi, w_ref[...], dn, preferred_element_type=jnp.float32)


def _merge_tables(t0t, t1t, w):
    last = pl.cdiv(_VOCAB, _CB) - 1
    lo = lambda i: (0, i)
    hi = lambda i: (0, jnp.minimum(i + _NMB, last))
    return pl.pallas_call(
        _merge_body,
        grid=(_NMB,),
        in_specs=[
            pl.BlockSpec((_D, _CB), lo),
            pl.BlockSpec((_D, _CB), lo),
            pl.BlockSpec((_D, _CB), hi),
            pl.BlockSpec((_D, _CB), hi),
            pl.BlockSpec((2 * _D, _D), lambda i: (0, 0)),
        ],
        out_specs=pl.BlockSpec((_CB, 2 * _D), lambda i: (i, 0)),
        out_shape=jax.ShapeDtypeStruct((_HALF, 2 * _D), jnp.float32),
    )(t0t, t1t, t0t, t1t, w)


_NW = 32
_BW = _B // _NW
_G = 128
_NG = _BW // _G


def _gather_merged(merged2, idx3, l0, nl):
    mesh = plsc.VectorSubcoreMesh(core_axis_name="c", subcore_axis_name="s")

    @functools.partial(
        pl.kernel,
        mesh=mesh,
        out_type=jax.ShapeDtypeStruct((nl * _B, _D), jnp.float32),
        scratch_types=[
            pltpu.VMEM((_NG, _G), jnp.int32),
            pltpu.VMEM((_BW, _D), jnp.float32),
            pltpu.SemaphoreType.DMA,
        ],
        compiler_params=pltpu.CompilerParams(use_tc_tiling_on_sc=False),
    )
    def _gather_kernel(m_hbm, idx_hbm, out_hbm, idx_v, rows_v, sem):
        wid = lax.axis_index("s") * 2 + lax.axis_index("c")
        b0 = wid * _BW

        def body(l, carry):
            pltpu.sync_copy(idx_hbm.at[l0 + l, pl.ds(wid * _NG, _NG)], idx_v)
            copies = []
            for j in range(_NG):
                copies.append(
                    pltpu.async_copy(
                        m_hbm.at[idx_v.at[j]],
                        rows_v.at[pl.ds(j * _G, _G)],
                        sem,
                    )
                )
            for c in copies:
                c.wait()
            pltpu.sync_copy(rows_v, out_hbm.at[pl.ds(l * _B + b0, _BW)])
            return carry

        lax.fori_loop(0, nl, body, 0)

    return _gather_kernel(merged2, idx3)


_HB = _BW // 2
_FC = 16


def _format_body(g_ref, out_ref):
    for c in range(_FC):
        t = g_ref[0, c].T
        out_ref[0, :, c * _BW : c * _BW + _HB] = t[: _D]
        out_ref[0, :, c * _BW + _HB : (c + 1) * _BW] = t[_D:]


def _format_body_alias(g_ref, prev_ref, out_ref):
    del prev_ref
    _format_body(g_ref, out_ref)


def _format_out(g4, l0, nl, prev=None):
    grid = (nl, _B // (_BW * _FC))
    in_specs = [pl.BlockSpec((1, _FC, _HB, 2 * _D), lambda l, c: (l, c, 0, 0))]
    args = (g4,)
    kwargs = {}
    body = _format_body
    if prev is not None:
        in_specs.append(pl.BlockSpec(memory_space=pl.ANY))
        args = (g4, prev)
        kwargs = dict(input_output_aliases={1: 0})
        body = _format_body_alias
    return pl.pallas_call(
        body,
        grid=grid,
        in_specs=in_specs,
        out_specs=pl.BlockSpec((1, _D, _BW * _FC), lambda l, c: (l0 + l, 0, c)),
        out_shape=jax.ShapeDtypeStruct((_L, _D, _B), jnp.float32),
        **kwargs,
    )(*args)


def kernel(x, table0, table1, P):
    w = jnp.concatenate(
        [(1.0 - _COEFF) * jnp.eye(_D, dtype=jnp.float32), _COEFF * P], axis=0
    )
    merged = _merge_tables(table0.T, table1.T, w)
    xt = x.T.astype(jnp.int32)
    idx = jnp.where(xt < _HALF, 2 * xt, 2 * (xt - _HALF) + 1)
    idx_perm = idx.reshape(_L, _B // _BW, 2, _HB).transpose(0, 1, 3, 2)
    idx3 = idx_perm.reshape(_L, _B // _G, _G)
    m2 = merged.reshape(2 * _HALF, _D)
    nl = _L // 2
    g_a = _gather_merged(m2, idx3, 0, nl)
    g_b = _gather_merged(m2, idx3, nl, _L - nl)
    out = _format_out(g_a.reshape(nl, _B // _BW, _HB, 2 * _D), 0, nl)
    out = _format_out(
        g_b.reshape(_L - nl, _B // _BW, _HB, 2 * _D), nl, _L - nl, prev=out
    )
    return jnp.transpose(out, (2, 0, 1))


# speedup vs baseline: 2.4028x; 1.0200x over previous
import functools

import jax
import jax.numpy as jnp
from jax import lax
from jax.experimental import pallas as pl
from jax.experimental.pallas import tpu as pltpu
from jax.experimental.pallas import tpu_sc as plsc

_VOCAB = 1000000
_D = 64
_B = 16384
_L = 50
_COEFF = 0.5
_N = _B * _L


_HALF = 524288
_CB = 8192
_NMB = _HALF // _CB


def _merge_body(t0lo_ref, t1lo_ref, t0hi_ref, t1hi_ref, w_ref, out_ref):
    alo = jnp.concatenate([t0lo_ref[...], t1lo_ref[...]], axis=0)
    ahi = jnp.concatenate([t0hi_ref[...], t1hi_ref[...]], axis=0)
    dn = (((0,), (0,)), ((), ()))
    out_ref[:, : _D] = lax.dot_general(
        alo, w_ref[...], dn, preferred_element_type=jnp.float32)
    out_ref[:, _D :] = lax.dot_general(
        ahi, w_ref[...], dn, preferred_element_type=jnp.float32)


def _merge_tables(t0t, t1t, w):
    last = pl.cdiv(_VOCAB, _CB) - 1
    lo = lambda i: (0, i)
    hi = lambda i: (0, jnp.minimum(i + _NMB, last))
    return pl.pallas_call(
        _merge_body,
        grid=(_NMB,),
        in_specs=[
            pl.BlockSpec((_D, _CB), lo),
            pl.BlockSpec((_D, _CB), lo),
            pl.BlockSpec((_D, _CB), hi),
            pl.BlockSpec((_D, _CB), hi),
            pl.BlockSpec((2 * _D, _D), lambda i: (0, 0)),
        ],
        out_specs=pl.BlockSpec((_CB, 2 * _D), lambda i: (i, 0)),
        out_shape=jax.ShapeDtypeStruct((_HALF, 2 * _D), jnp.float32),
    )(t0t, t1t, t0t, t1t, w)


_NW = 32
_BW = _B // _NW
_G = 128
_NG = _BW // _G


def _gather_merged(merged2, idx3, l0, nl):
    mesh = plsc.VectorSubcoreMesh(core_axis_name="c", subcore_axis_name="s")

    @functools.partial(
        pl.kernel,
        mesh=mesh,
        out_type=jax.ShapeDtypeStruct((nl * _B, _D), jnp.float32),
        scratch_types=[
            pltpu.VMEM((_NG, _G), jnp.int32),
            pltpu.VMEM((_BW, _D), jnp.float32),
            pltpu.SemaphoreType.DMA,
        ],
        compiler_params=pltpu.CompilerParams(use_tc_tiling_on_sc=False),
    )
    def _gather_kernel(m_hbm, idx_hbm, out_hbm, idx_v, rows_v, sem):
        wid = lax.axis_index("s") * 2 + lax.axis_index("c")
        b0 = wid * _BW

        def body(l, carry):
            pltpu.sync_copy(idx_hbm.at[l0 + l, pl.ds(wid * _NG, _NG)], idx_v)
            copies = []
            for j in range(_NG):
                copies.append(
                    pltpu.async_copy(
                        m_hbm.at[idx_v.at[j]],
                        rows_v.at[pl.ds(j * _G, _G)],
                        sem,
                    )
                )
            for c in copies:
                c.wait()
            pltpu.sync_copy(rows_v, out_hbm.at[pl.ds(l * _B + b0, _BW)])
            return carry

        lax.fori_loop(0, nl, body, 0)

    return _gather_kernel(merged2, idx3)


_HB = _BW // 2
_FC = 16


def _format_body(g_ref, out_ref):
    for c in range(_FC):
        t = g_ref[0, c].T
        out_ref[0, :, c * _BW : c * _BW + _HB] = t[: _D]
        out_ref[0, :, c * _BW + _HB : (c + 1) * _BW] = t[_D:]


def _format_body_alias(g_ref, prev_ref, out_ref):
    del prev_ref
    _format_body(g_ref, out_ref)


def _format_out(g4, l0, nl, prev=None):
    grid = (nl, _B // (_BW * _FC))
    in_specs = [pl.BlockSpec((1, _FC, _HB, 2 * _D), lambda l, c: (l, c, 0, 0))]
    args = (g4,)
    kwargs = {}
    body = _format_body
    if prev is not None:
        in_specs.append(pl.BlockSpec(memory_space=pl.ANY))
        args = (g4, prev)
        kwargs = dict(input_output_aliases={1: 0})
        body = _format_body_alias
    return pl.pallas_call(
        body,
        grid=grid,
        in_specs=in_specs,
        out_specs=pl.BlockSpec((1, _D, _BW * _FC), lambda l, c: (l0 + l, 0, c)),
        out_shape=jax.ShapeDtypeStruct((_L, _D, _B), jnp.float32),
        **kwargs,
    )(*args)


def kernel(x, table0, table1, P):
    w = jnp.concatenate(
        [(1.0 - _COEFF) * jnp.eye(_D, dtype=jnp.float32), _COEFF * P], axis=0
    )
    merged = _merge_tables(table0.T, table1.T, w)
    xt = x.T.astype(jnp.int32)
    idx = jnp.where(xt < _HALF, 2 * xt, 2 * (xt - _HALF) + 1)
    idx_perm = idx.reshape(_L, _B // _BW, 2, _HB).transpose(0, 1, 3, 2)
    idx3 = idx_perm.reshape(_L, _B // _G, _G)
    m2 = merged.reshape(2 * _HALF, _D)
    slices = (13, 13, 12, 12)
    gs = []
    l0 = 0
    for nl in slices:
        gs.append((l0, nl, _gather_merged(m2, idx3, l0, nl)))
        l0 += nl
    out = None
    for l0, nl, g in gs:
        out = _format_out(
            g.reshape(nl, _B // _BW, _HB, 2 * _D), l0, nl, prev=out
        )
    return jnp.transpose(out, (2, 0, 1))
